# baseline (speedup 1.0000x reference)
import functools

import jax
import jax.numpy as jnp
from jax import lax
from jax.experimental import pallas as pl
from jax.experimental.pallas import tpu as pltpu
from jax.experimental.pallas import tpu_sc as plsc

K = 8192
D = 64
B = 16
T = 1024
N = B * T
TBLK = 512
NB = N // TBLK

NC = 2
NS = 16
NW = NC * NS
BPW = N // NW
CH = 128
NCH = BPW // CH


def _argmin_body(xf_ref, xsq_ref, csq_ref, cb2_ref, idx_ref, dsum_ref):
    x_blk = xf_ref[...]
    dots2 = lax.dot_general(
        x_blk, cb2_ref[...],
        dimension_numbers=(((1,), (1,)), ((), ())),
        preferred_element_type=jnp.float32)
    dist = (xsq_ref[...] - dots2) + csq_ref[...]
    m = jnp.min(dist, axis=1)
    idx_ref[0, 0, :] = jnp.argmin(dist, axis=1).astype(jnp.int32)
    dsum_ref[...] = jnp.broadcast_to(jnp.sum(m), (1, 1, TBLK))


def _argmin_call(xf, x_sq, c_sq, cb2):
    nb = xf.shape[0] // TBLK
    return pl.pallas_call(
        _argmin_body,
        grid=(nb,),
        in_specs=[
            pl.BlockSpec((TBLK, D), lambda i: (i, 0)),
            pl.BlockSpec((TBLK, 1), lambda i: (i, 0)),
            pl.BlockSpec((1, K), lambda i: (0, 0)),
            pl.BlockSpec((K, D), lambda i: (0, 0)),
        ],
        out_specs=[
            pl.BlockSpec((1, 1, TBLK), lambda i: (i, 0, 0)),
            pl.BlockSpec((1, 1, TBLK), lambda i: (i, 0, 0)),
        ],
        out_shape=[
            jax.ShapeDtypeStruct((nb, 1, TBLK), jnp.int32),
            jax.ShapeDtypeStruct((nb, 1, TBLK), jnp.float32),
        ],
    )(xf, x_sq, c_sq, cb2)


DPAD = 128


def _sc_gather(codebook_padded, idx2d):
    n = idx2d.shape[0] * CH
    bpw = n // NW
    nch = bpw // CH
    mesh = plsc.VectorSubcoreMesh(core_axis_name="c", subcore_axis_name="s")

    @functools.partial(
        pl.kernel, mesh=mesh,
        out_type=jax.ShapeDtypeStruct((n, DPAD), jnp.float32),
        scratch_types=[
            pltpu.VMEM((nch, CH), jnp.int32),
            pltpu.VMEM((bpw, DPAD), jnp.float32),
            pltpu.SemaphoreType.DMA,
        ],
    )
    def gather_k(table_hbm, idx_hbm, out_hbm, idx_v, rows_v, sem):
        wid = lax.axis_index("s") * NC + lax.axis_index("c")
        pltpu.sync_copy(idx_hbm.at[pl.ds(wid * nch, nch)], idx_v)
        copies = [
            pltpu.async_copy(table_hbm.at[idx_v.at[j]],
                             rows_v.at[pl.ds(j * CH, CH)], sem)
            for j in range(nch)
        ]
        for c in copies:
            c.wait()
        pltpu.sync_copy(rows_v, out_hbm.at[pl.ds(wid * bpw, bpw)])

    return gather_k(codebook_padded, idx2d)


def kernel(x, codebook):
    xf = jnp.transpose(x, (0, 2, 1)).reshape(N, D)
    x_sq = jnp.sum(xf * xf, axis=-1, keepdims=True)
    c_sq = jnp.sum(codebook * codebook, axis=-1)[None, :]

    cb2 = codebook + codebook
    cb_pad = jnp.pad(codebook, ((0, 0), (0, DPAD - D)))

    H = N // 2
    idxs, dsums, qs = [], [], []
    for h in range(2):
        sl = slice(h * H, (h + 1) * H)
        idx3, dsum = _argmin_call(xf[sl], x_sq[sl], c_sq, cb2)
        q = _sc_gather(cb_pad, idx3.reshape(H // CH, CH))
        idxs.append(idx3)
        dsums.append(dsum)
        qs.append(q)

    indices = jnp.concatenate(idxs).reshape(B, T)
    q = jnp.concatenate(qs)[:, :D]
    quantized_out = jnp.transpose(q.reshape(B, T, D), (0, 2, 1))

    commit_loss = (0.25 / (N * D)) * (
        jnp.sum(dsums[0][:, 0, 0]) + jnp.sum(dsums[1][:, 0, 0]))
    return (quantized_out, indices, commit_loss)


# speedup vs baseline: 1.0951x; 1.0951x over previous
import functools

import jax
import jax.numpy as jnp
from jax import lax
from jax.experimental import pallas as pl
from jax.experimental.pallas import tpu as pltpu
from jax.experimental.pallas import tpu_sc as plsc

K = 8192
D = 64
B = 16
T = 1024
N = B * T
TBLK = 512
NB = N // TBLK

NC = 2
NS = 16
NW = NC * NS
BPW = N // NW
CH = 128
NCH = BPW // CH


def _argmin_body(xf_ref, xsq_ref, csq_ref, cb2_ref, idx_ref, dsum_ref):
    x_blk = xf_ref[...]
    dots2 = lax.dot_general(
        x_blk, cb2_ref[...],
        dimension_numbers=(((1,), (1,)), ((), ())),
        preferred_element_type=jnp.float32)
    dist = (xsq_ref[...] - dots2) + csq_ref[...]
    m = jnp.min(dist, axis=1)
    idx_ref[0, 0, :] = jnp.argmin(dist, axis=1).astype(jnp.int32)
    dsum_ref[...] = jnp.broadcast_to(jnp.sum(m), (1, 1, TBLK))


def _argmin_call(xf, x_sq, c_sq, cb2):
    nb = xf.shape[0] // TBLK
    return pl.pallas_call(
        _argmin_body,
        grid=(nb,),
        in_specs=[
            pl.BlockSpec((TBLK, D), lambda i: (i, 0)),
            pl.BlockSpec((TBLK, 1), lambda i: (i, 0)),
            pl.BlockSpec((1, K), lambda i: (0, 0)),
            pl.BlockSpec((K, D), lambda i: (0, 0)),
        ],
        out_specs=[
            pl.BlockSpec((1, 1, TBLK), lambda i: (i, 0, 0)),
            pl.BlockSpec((1, 1, TBLK), lambda i: (i, 0, 0)),
        ],
        out_shape=[
            jax.ShapeDtypeStruct((nb, 1, TBLK), jnp.int32),
            jax.ShapeDtypeStruct((nb, 1, TBLK), jnp.float32),
        ],
    )(xf, x_sq, c_sq, cb2)


DPAD = 128


def _sc_gather(codebook_padded, idx2d):
    n = idx2d.shape[0] * CH
    bpw = n // NW
    nch = bpw // CH
    mesh = plsc.VectorSubcoreMesh(core_axis_name="c", subcore_axis_name="s")

    @functools.partial(
        pl.kernel, mesh=mesh,
        out_type=jax.ShapeDtypeStruct((n, DPAD), jnp.float32),
        scratch_types=[
            pltpu.VMEM((nch, CH), jnp.int32),
            pltpu.VMEM((bpw, DPAD), jnp.float32),
            pltpu.SemaphoreType.DMA,
        ],
    )
    def gather_k(table_hbm, idx_hbm, out_hbm, idx_v, rows_v, sem):
        wid = lax.axis_index("s") * NC + lax.axis_index("c")
        pltpu.sync_copy(idx_hbm.at[pl.ds(wid * nch, nch)], idx_v)
        copies = [
            pltpu.async_copy(table_hbm.at[idx_v.at[j]],
                             rows_v.at[pl.ds(j * CH, CH)], sem)
            for j in range(nch)
        ]
        for c in copies:
            c.wait()
        pltpu.sync_copy(rows_v, out_hbm.at[pl.ds(wid * bpw, bpw)])

    return gather_k(codebook_padded, idx2d)


def kernel(x, codebook):
    xf = jnp.transpose(x, (0, 2, 1)).reshape(N, D)
    x_sq = jnp.sum(xf * xf, axis=-1, keepdims=True)
    c_sq = jnp.sum(codebook * codebook, axis=-1)[None, :]

    cb2 = codebook + codebook
    cb_pad = jnp.pad(codebook, ((0, 0), (0, DPAD - D)))

    idx3, dsum = _argmin_call(xf, x_sq, c_sq, cb2)
    indices = idx3.reshape(B, T)

    q = _sc_gather(cb_pad, idx3.reshape(N // CH, CH))
    quantized_out = jnp.transpose(q[:, :D].reshape(B, T, D), (0, 2, 1))

    commit_loss = (0.25 / (N * D)) * jnp.sum(dsum[:, 0, 0])
    return (quantized_out, indices, commit_loss)
